# Initial kernel scaffold; baseline (speedup 1.0000x reference)
import jax
import jax.numpy as jnp
from jax.experimental import pallas as pl


def kernel(inputs, W):
    raise NotImplementedError("write your pallas kernel here")


# speedup vs baseline: 1.2263x; 1.2263x over previous
import jax
import jax.numpy as jnp
from jax import lax
from jax.experimental import pallas as pl

_NUM_EMB = 1024
_DIM = 64
_ROWS = 16 * 32 * 32
_BLK = 2048
_COST = 0.25


def _vq_body(x_ref, wt_ref, w_ref, idx_ref, q_ref, dsum_ref):
    x = x_ref[...]
    wt = wt_ref[...]
    xs = jnp.sum(x * x, axis=1, keepdims=True)
    ws = jnp.sum(wt * wt, axis=0, keepdims=True)
    mm = lax.dot_general(x, wt, (((1,), (0,)), ((), ())),
                         preferred_element_type=jnp.float32)
    d = (xs + ws) - 2.0 * mm
    dmin = jnp.min(d, axis=1, keepdims=True)
    cix = lax.broadcasted_iota(jnp.int32, d.shape, 1)
    idx = jnp.min(jnp.where(d == dmin, cix, _NUM_EMB), axis=1, keepdims=True)
    idx_ref[...] = idx
    enc = (cix == idx).astype(jnp.float32)
    q_ref[...] = lax.dot_general(enc, w_ref[...], (((1,), (0,)), ((), ())),
                                 preferred_element_type=jnp.float32)

    @pl.when(pl.program_id(0) == 0)
    def _init():
        dsum_ref[...] = jnp.zeros_like(dsum_ref)

    dsum_ref[...] += jnp.sum(dmin).reshape(1, 1)


def kernel(inputs, W):
    x = jnp.transpose(inputs, (0, 2, 3, 1))
    xf = x.reshape(_ROWS, _DIM)
    wt = W.T
    idx, q, dsum = pl.pallas_call(
        _vq_body,
        grid=(_ROWS // _BLK,),
        in_specs=[
            pl.BlockSpec((_BLK, _DIM), lambda i: (i, 0)),
            pl.BlockSpec((_DIM, _NUM_EMB), lambda i: (0, 0)),
            pl.BlockSpec((_NUM_EMB, _DIM), lambda i: (0, 0)),
        ],
        out_specs=[
            pl.BlockSpec((_BLK, 1), lambda i: (i, 0)),
            pl.BlockSpec((_BLK, _DIM), lambda i: (i, 0)),
            pl.BlockSpec((1, 1), lambda i: (0, 0)),
        ],
        out_shape=[
            jax.ShapeDtypeStruct((_ROWS, 1), jnp.int32),
            jax.ShapeDtypeStruct((_ROWS, _DIM), jnp.float32),
            jax.ShapeDtypeStruct((1, 1), jnp.float32),
        ],
    )(xf, wt, W)
    loss = (1.0 + _COST) * dsum[0, 0] / (_ROWS * _DIM)
    qst = q.reshape(16, 32, 32, _DIM).transpose(0, 3, 1, 2)
    return qst, loss, idx
